# speedup vs baseline: 1.7437x; 1.7437x over previous
import functools

import jax
import jax.numpy as jnp
from jax import lax
from jax.experimental import pallas as pl
from jax.experimental.pallas import tpu as pltpu
from jax.experimental.pallas import tpu_sc as plsc

N_NODES = 100000
N_FEATURES = 128
CHUNK = 128
NW = 32
NCHUNKS = -(-N_NODES // CHUNK)
ITERS = -(-NCHUNKS // NW)
LAST_START = N_NODES - CHUNK


def _emb_kernel(idx_hbm, table_hbm, out_hbm, idx_v, rows_v, sem):
    wid = lax.axis_index("s") * 2 + lax.axis_index("c")

    def body(t, carry):
        c = wid + t * NW
        start = jnp.minimum(c * CHUNK, LAST_START)
        pltpu.sync_copy(idx_hbm.at[pl.ds(start, CHUNK)], idx_v)
        pltpu.async_copy(table_hbm.at[idx_v], rows_v, sem).wait()
        pltpu.sync_copy(rows_v, out_hbm.at[pl.ds(start, CHUNK)])
        return carry

    lax.fori_loop(0, ITERS, body, 0)


@jax.jit
def _emb(node_specie, table):
    mesh = plsc.VectorSubcoreMesh(core_axis_name="c", subcore_axis_name="s")
    f = functools.partial(
        pl.kernel,
        mesh=mesh,
        out_type=jax.ShapeDtypeStruct((N_NODES, N_FEATURES), jnp.float32),
        scratch_types=[
            pltpu.VMEM((CHUNK,), jnp.int32),
            pltpu.VMEM((CHUNK, N_FEATURES), jnp.float32),
            pltpu.SemaphoreType.DMA,
        ],
    )(_emb_kernel)
    return f(node_specie, table)


def kernel(node_specie, embeddings_0):
    table = embeddings_0.reshape(embeddings_0.shape[0], N_FEATURES)
    out = _emb(node_specie, table)
    return out.reshape(N_NODES, N_FEATURES, 1)


# speedup vs baseline: 1.8099x; 1.0379x over previous
import functools

import jax
import jax.numpy as jnp
from jax import lax
from jax.experimental import pallas as pl
from jax.experimental.pallas import tpu as pltpu
from jax.experimental.pallas import tpu_sc as plsc

N_NODES = 100000
N_FEATURES = 128
CHUNK = 128
NW = 32
CPW = 25
ROWS_PW = CPW * CHUNK
WSTRIDE = 3128
LAST_BASE = N_NODES - ROWS_PW
NBUF = 2


def _emb_kernel(idx_hbm, table_hbm, out_hbm,
                idx_v, buf0, buf1, gsem0, gsem1, osem0, osem1):
    wid = lax.axis_index("s") * 2 + lax.axis_index("c")
    base = jnp.minimum(wid * WSTRIDE, LAST_BASE)
    pltpu.sync_copy(idx_hbm.at[pl.ds(base, ROWS_PW)], idx_v)

    bufs = (buf0, buf1)
    gsems = (gsem0, gsem1)
    osems = (osem0, osem1)

    def gather(t, b):
        pltpu.async_copy(
            table_hbm.at[idx_v.at[pl.ds(t * CHUNK, CHUNK)]], bufs[b], gsems[b])

    def gwait(b):
        pltpu.make_async_copy(
            table_hbm.at[idx_v.at[pl.ds(0, CHUNK)]], bufs[b], gsems[b]).wait()

    def outcopy(t, b):
        pltpu.async_copy(
            bufs[b], out_hbm.at[pl.ds(base + t * CHUNK, CHUNK)], osems[b])

    def owait(b):
        pltpu.make_async_copy(
            bufs[b], out_hbm.at[pl.ds(base, CHUNK)], osems[b]).wait()

    gather(0, 0)
    gather(1, 1)

    def pair(p, carry):
        for b in range(NBUF):
            t = p * NBUF + b
            gwait(b)
            outcopy(t, b)
            owait(b)
            gather(jnp.minimum(t + NBUF, CPW - 1), b)
        return carry

    lax.fori_loop(0, (CPW - 1) // NBUF, pair, 0)

    gwait(0)
    outcopy(CPW - 1, 0)
    gwait(1)
    owait(0)


@jax.jit
def _emb(node_specie, table):
    mesh = plsc.VectorSubcoreMesh(core_axis_name="c", subcore_axis_name="s")
    f = functools.partial(
        pl.kernel,
        mesh=mesh,
        out_type=jax.ShapeDtypeStruct((N_NODES, N_FEATURES), jnp.float32),
        scratch_types=[
            pltpu.VMEM((ROWS_PW,), jnp.int32),
            pltpu.VMEM((CHUNK, N_FEATURES), jnp.float32),
            pltpu.VMEM((CHUNK, N_FEATURES), jnp.float32),
            pltpu.SemaphoreType.DMA,
            pltpu.SemaphoreType.DMA,
            pltpu.SemaphoreType.DMA,
            pltpu.SemaphoreType.DMA,
        ],
    )(_emb_kernel)
    return f(node_specie, table)


def kernel(node_specie, embeddings_0):
    table = embeddings_0.reshape(embeddings_0.shape[0], N_FEATURES)
    out = _emb(node_specie, table)
    return out.reshape(N_NODES, N_FEATURES, 1)
